# baseline (speedup 1.0000x reference)
import jax
import jax.numpy as jnp
from jax import lax
from jax.experimental import pallas as pl

MARGIN = 0.25
GAMMA = 256.0

_BLK_C = 2048
_STRIP_R = 8
_UNROLL = 16


def _circle_loss_block(labels_ref, x_ref, o_ref):
    j = pl.program_id(0)
    col = jax.lax.broadcasted_iota(jnp.int32, (_STRIP_R, _BLK_C), 1) + j * _BLK_C

    def strip_group(i, _):
        for u in range(_UNROLL):
            r = (i * _UNROLL + u) * _STRIP_R
            x = x_ref[pl.ds(r, _STRIP_R), :]
            cosu = jnp.minimum(x, 1.0)
            cm = cosu * GAMMA
            alpha_g = jnp.maximum(cm + (GAMMA * MARGIN), 0.0)
            neg = alpha_g * (cosu - MARGIN)
            pos = jnp.maximum(cm, -GAMMA)
            lab = labels_ref[pl.ds(r, _STRIP_R), :]
            o_ref[pl.ds(r, _STRIP_R), :] = jnp.where(col == lab, pos, neg)
        return _

    nstrip = x_ref.shape[0] // _STRIP_R
    lax.fori_loop(0, nstrip // _UNROLL, strip_group, None)


def kernel(cos_theta, labels):
    b, c = cos_theta.shape
    labels2d = labels.astype(jnp.int32).reshape(b, 1)
    grid = (pl.cdiv(c, _BLK_C),)
    return pl.pallas_call(
        _circle_loss_block,
        grid=grid,
        in_specs=[
            pl.BlockSpec((b, 1), lambda j: (0, 0)),
            pl.BlockSpec((b, _BLK_C), lambda j: (0, j)),
        ],
        out_specs=pl.BlockSpec((b, _BLK_C), lambda j: (0, j)),
        out_shape=jax.ShapeDtypeStruct((b, c), jnp.float32),
    )(labels2d, cos_theta)


# speedup vs baseline: 1.0000x; 1.0000x over previous
import jax
import jax.numpy as jnp
from jax import lax
from jax.experimental import pallas as pl

MARGIN = 0.25
GAMMA = 256.0

_BLK_C = 3072
_STRIP_R = 8
_UNROLL = 16


def _circle_loss_block(labels_ref, x_ref, o_ref):
    j = pl.program_id(0)
    col = jax.lax.broadcasted_iota(jnp.int32, (_STRIP_R, _BLK_C), 1) + j * _BLK_C

    def strip_group(i, _):
        for u in range(_UNROLL):
            r = (i * _UNROLL + u) * _STRIP_R
            x = x_ref[pl.ds(r, _STRIP_R), :]
            cosu = jnp.minimum(x, 1.0)
            cm = cosu * GAMMA
            alpha_g = jnp.maximum(cm + (GAMMA * MARGIN), 0.0)
            neg = alpha_g * (cosu - MARGIN)
            pos = jnp.maximum(cm, -GAMMA)
            lab = labels_ref[pl.ds(r, _STRIP_R), :]
            o_ref[pl.ds(r, _STRIP_R), :] = jnp.where(col == lab, pos, neg)
        return _

    nstrip = x_ref.shape[0] // _STRIP_R
    lax.fori_loop(0, nstrip // _UNROLL, strip_group, None)


def kernel(cos_theta, labels):
    b, c = cos_theta.shape
    labels2d = labels.astype(jnp.int32).reshape(b, 1)
    grid = (pl.cdiv(c, _BLK_C),)
    return pl.pallas_call(
        _circle_loss_block,
        grid=grid,
        in_specs=[
            pl.BlockSpec((b, 1), lambda j: (0, 0)),
            pl.BlockSpec((b, _BLK_C), lambda j: (0, j)),
        ],
        out_specs=pl.BlockSpec((b, _BLK_C), lambda j: (0, j)),
        out_shape=jax.ShapeDtypeStruct((b, c), jnp.float32),
    )(labels2d, cos_theta)


# speedup vs baseline: 1.0023x; 1.0022x over previous
import jax
import jax.numpy as jnp
from jax import lax
from jax.experimental import pallas as pl

MARGIN = 0.25
GAMMA = 256.0

_BLK_C = 2048
_STRIP_R = 8
_UNROLL = 32


def _circle_loss_block(labels_ref, x_ref, o_ref):
    j = pl.program_id(0)
    col = jax.lax.broadcasted_iota(jnp.int32, (_STRIP_R, _BLK_C), 1) + j * _BLK_C

    def strip_group(i, _):
        for u in range(_UNROLL):
            r = (i * _UNROLL + u) * _STRIP_R
            x = x_ref[pl.ds(r, _STRIP_R), :]
            cosu = jnp.minimum(x, 1.0)
            cm = cosu * GAMMA
            alpha_g = jnp.maximum(cm + (GAMMA * MARGIN), 0.0)
            neg = alpha_g * (cosu - MARGIN)
            pos = jnp.maximum(cm, -GAMMA)
            lab = labels_ref[pl.ds(r, _STRIP_R), :]
            o_ref[pl.ds(r, _STRIP_R), :] = jnp.where(col == lab, pos, neg)
        return _

    nstrip = x_ref.shape[0] // _STRIP_R
    lax.fori_loop(0, nstrip // _UNROLL, strip_group, None)


def kernel(cos_theta, labels):
    b, c = cos_theta.shape
    labels2d = labels.astype(jnp.int32).reshape(b, 1)
    grid = (pl.cdiv(c, _BLK_C),)
    return pl.pallas_call(
        _circle_loss_block,
        grid=grid,
        in_specs=[
            pl.BlockSpec((b, 1), lambda j: (0, 0)),
            pl.BlockSpec((b, _BLK_C), lambda j: (0, j)),
        ],
        out_specs=pl.BlockSpec((b, _BLK_C), lambda j: (0, j)),
        out_shape=jax.ShapeDtypeStruct((b, c), jnp.float32),
    )(labels2d, cos_theta)
